# baseline (speedup 1.0000x reference)
import functools

import jax
import jax.numpy as jnp
from jax import lax
from jax.experimental import pallas as pl
from jax.experimental.pallas import tpu as pltpu
from jax.experimental.pallas import tpu_sc as plsc

_D = 64
_NC = 2
_NS = 16
_NW = _NC * _NS
_LANES = 16


@functools.lru_cache(maxsize=None)
def _build_gather(B: int, S: int, V2: int):
    b_per_w = B // _NW
    ch = 256
    halves = b_per_w // ch
    n_ch = S * halves

    mesh = plsc.VectorSubcoreMesh(core_axis_name="c", subcore_axis_name="s")

    @functools.partial(
        pl.kernel,
        mesh=mesh,
        out_type=jax.ShapeDtypeStruct((S, _D, B), jnp.float32),
        compiler_params=pltpu.CompilerParams(
            use_tc_tiling_on_sc=True, needs_layout_passes=False),
        scratch_types=[
            pltpu.VMEM((S, b_per_w), jnp.int32),
            pltpu.VMEM((2, ch, 2 * _D), jnp.float32),
            pltpu.VMEM((2, _D, ch), jnp.float32),
            pltpu.SemaphoreType.DMA,
            pltpu.SemaphoreType.DMA,
            pltpu.SemaphoreType.DMA,
            pltpu.SemaphoreType.DMA,
        ],
    )
    def gather_kernel(ids_hbm, table_hbm, out_hbm, ids_v, gbuf, tbuf,
                      gs0, gs1, ss0, ss1):
        wid = lax.axis_index("s") * _NC + lax.axis_index("c")
        wb0 = wid * b_per_w
        gsems = (gs0, gs1)
        ssems = (ss0, ss1)
        iota = lax.iota(jnp.int32, _LANES)
        n_g = ch // _LANES

        pltpu.sync_copy(ids_hbm.at[:, pl.ds(wb0, b_per_w)], ids_v)

        def gather_descs(k, b):
            s, half = k >> 1, k & 1
            return [
                pltpu.make_async_copy(
                    table_hbm.at[ids_v.at[s].at[
                        pl.ds(half * ch + j * 128, 128)]],
                    gbuf.at[b].at[pl.ds(j * 128, 128)],
                    gsems[b],
                )
                for j in range(ch // 128)
            ]

        def store_desc(k, b):
            s, half = k >> 1, k & 1
            return pltpu.make_async_copy(
                tbuf.at[b],
                out_hbm.at[s, :, pl.ds(wb0 + half * ch, ch)],
                ssems[b],
            )

        def transpose_chunk(k, b):
            rowi = [iota + (g * _LANES) for g in range(n_g)]

            @pl.loop(0, _D)
            def _d(d):
                dcol = jnp.full((_LANES,), 0, jnp.int32) + d
                for g in range(n_g):
                    vals = plsc.load_gather(gbuf.at[b], [rowi[g], dcol])
                    tbuf[b, d, pl.ds(g * _LANES, _LANES)] = vals

        def fire_gather(k, b):
            for dsc in gather_descs(k, b):
                dsc.start()

        fire_gather(0, 0)
        fire_gather(1, 1)

        @pl.loop(0, n_ch, step=2)
        def _pair(t):
            for b in range(2):
                k = t + b
                for dsc in gather_descs(k, b):
                    dsc.wait()

                @pl.when(k >= 2)
                def _():
                    store_desc(k - 2, b).wait()

                transpose_chunk(k, b)
                store_desc(k, b).start()

                @pl.when(k + 2 < n_ch)
                def _():
                    fire_gather(k + 2, b)

        store_desc(n_ch - 2, 0).wait()
        store_desc(n_ch - 1, 1).wait()

    return gather_kernel


def kernel(token_ids, embedding):
    b, s = token_ids.shape
    v, d = embedding.shape
    ids_t = token_ids.T.astype(jnp.int32)
    table2 = jnp.pad(embedding, ((0, 0), (0, d)))
    out3 = _build_gather(b, s, v)(ids_t, table2)
    return out3.transpose(2, 0, 1)


# speedup vs baseline: 1.4153x; 1.4153x over previous
import functools

import jax
import jax.numpy as jnp
from jax import lax
from jax.experimental import pallas as pl
from jax.experimental.pallas import tpu as pltpu
from jax.experimental.pallas import tpu_sc as plsc

_D = 64
_NC = 2
_NS = 16
_NW = _NC * _NS
_LANES = 16


@functools.lru_cache(maxsize=None)
def _build_gather(B: int, S: int, V2: int):
    b_per_w = B // _NW
    ch = 256
    halves = b_per_w // ch
    n_ch = S * halves

    mesh = plsc.VectorSubcoreMesh(core_axis_name="c", subcore_axis_name="s")

    @functools.partial(
        pl.kernel,
        mesh=mesh,
        out_type=jax.ShapeDtypeStruct((S, _D, B), jnp.float32),
        compiler_params=pltpu.CompilerParams(
            use_tc_tiling_on_sc=True, needs_layout_passes=False),
        scratch_types=[
            pltpu.VMEM((S, b_per_w), jnp.int32),
            pltpu.VMEM((2, ch, 2 * _D), jnp.float32),
            pltpu.VMEM((2, _D, ch), jnp.float32),
            pltpu.SemaphoreType.DMA,
            pltpu.SemaphoreType.DMA,
            pltpu.SemaphoreType.DMA,
            pltpu.SemaphoreType.DMA,
        ],
    )
    def gather_kernel(ids_hbm, table_hbm, out_hbm, ids_v, gbuf, tbuf,
                      gs0, gs1, ss0, ss1):
        wid = lax.axis_index("s") * _NC + lax.axis_index("c")
        wb0 = wid * b_per_w
        gsems = (gs0, gs1)
        ssems = (ss0, ss1)
        iota = lax.iota(jnp.int32, _LANES)
        n_g = ch // _LANES

        pltpu.sync_copy(ids_hbm.at[:, pl.ds(wb0, b_per_w)], ids_v)

        def gather_descs(k, b):
            s, half = k >> 1, k & 1
            return [
                pltpu.make_async_copy(
                    table_hbm.at[ids_v.at[s].at[
                        pl.ds(half * ch + j * 128, 128)]],
                    gbuf.at[b].at[pl.ds(j * 128, 128)],
                    gsems[b],
                )
                for j in range(ch // 128)
            ]

        def store_desc(k, b):
            s, half = k >> 1, k & 1
            return pltpu.make_async_copy(
                tbuf.at[b],
                out_hbm.at[s, :, pl.ds(wb0 + half * ch, ch)],
                ssems[b],
            )

        def transpose_chunk(k, b):
            diag = [(iota + p) & (_LANES - 1) for p in range(_LANES)]

            @pl.loop(0, n_g)
            def _g(g):
                rowg = iota + g * _LANES
                for dg in range(_D // _LANES):
                    d0 = dg * _LANES
                    for p in range(_LANES):
                        dcol = diag[p] + d0
                        vals = plsc.load_gather(gbuf.at[b], [rowg, dcol])
                        plsc.store_scatter(tbuf.at[b], [dcol, rowg], vals)

        def fire_gather(k, b):
            for dsc in gather_descs(k, b):
                dsc.start()

        fire_gather(0, 0)
        fire_gather(1, 1)

        @pl.loop(0, n_ch, step=2)
        def _pair(t):
            for b in range(2):
                k = t + b
                for dsc in gather_descs(k, b):
                    dsc.wait()

                @pl.when(k >= 2)
                def _():
                    store_desc(k - 2, b).wait()

                transpose_chunk(k, b)
                store_desc(k, b).start()

                @pl.when(k + 2 < n_ch)
                def _():
                    fire_gather(k + 2, b)

        store_desc(n_ch - 2, 0).wait()
        store_desc(n_ch - 1, 1).wait()

    return gather_kernel


def kernel(token_ids, embedding):
    b, s = token_ids.shape
    v, d = embedding.shape
    ids_t = token_ids.T.astype(jnp.int32)
    table2 = jnp.pad(embedding, ((0, 0), (0, d)))
    out3 = _build_gather(b, s, v)(ids_t, table2)
    return out3.transpose(2, 0, 1)
